# baseline (speedup 1.0000x reference)
import jax
import jax.numpy as jnp
from jax.experimental import pallas as pl

D_MODEL = 1024
D_FF = 4096
N_BLOCK = 32
D1 = 128
D_LOWRANK = 64
TB = 512


def _ctrl_body(x_ref, m1_ref, m2_ref, mb_ref, am_ref):
    xt = x_ref[...]
    t1 = jnp.dot(xt, m1_ref[...], preferred_element_type=jnp.float32)
    lg = jnp.dot(t1, m2_ref[...], preferred_element_type=jnp.float32)
    lg = lg + mb_ref[...]
    m = lg[:, 0:D1]
    am = jnp.zeros((TB, D1), dtype=jnp.int32)
    for y in range(1, N_BLOCK):
        ly = lg[:, y * D1:(y + 1) * D1]
        gt = ly > m
        am = jnp.where(gt, y, am)
        m = jnp.where(gt, ly, m)
    am_ref[...] = am


def _main_body(x_ref, am_ref, w1_ref, w2_ref, b2_ref, out_ref):
    xt = x_ref[...]
    am = am_ref[...]
    midf = jnp.dot(xt, w1_ref[...], preferred_element_type=jnp.float32)
    zeros = jnp.zeros((TB, D1), dtype=jnp.float32)
    pieces = []
    for y in range(N_BLOCK):
        my = midf[:, y * D1:(y + 1) * D1]
        pieces.append(jnp.where(am == y, jnp.maximum(my, 0.0), zeros))
    hsel = jnp.concatenate(pieces, axis=1)
    out = jnp.dot(hsel, w2_ref[...], preferred_element_type=jnp.float32)
    out_ref[...] = out + b2_ref[...]


@jax.jit
def kernel(x, m1, m2, mb, w1, w2, b2):
    B, S, _ = x.shape
    T = B * S
    xf = x.reshape(T, D_MODEL)
    xb = xf.astype(jnp.bfloat16)
    m2f = m2.transpose(0, 2, 1).reshape(D_LOWRANK, D_FF)
    mbf = mb.transpose(1, 0).reshape(1, D_FF)
    w1f = w1.transpose(0, 2, 1).reshape(D_MODEL, D_FF).astype(jnp.bfloat16)
    w2f = w2.reshape(D_FF, D_MODEL)
    b2f = b2.reshape(1, D_MODEL)

    grid = (T // TB,)
    am = pl.pallas_call(
        _ctrl_body,
        grid=grid,
        in_specs=[
            pl.BlockSpec((TB, D_MODEL), lambda i: (i, 0)),
            pl.BlockSpec((D_MODEL, D_LOWRANK), lambda i: (0, 0)),
            pl.BlockSpec((D_LOWRANK, D_FF), lambda i: (0, 0)),
            pl.BlockSpec((1, D_FF), lambda i: (0, 0)),
        ],
        out_specs=pl.BlockSpec((TB, D1), lambda i: (i, 0)),
        out_shape=jax.ShapeDtypeStruct((T, D1), jnp.int32),
    )(xf, m1, m2f, mbf)
    out = pl.pallas_call(
        _main_body,
        grid=grid,
        in_specs=[
            pl.BlockSpec((TB, D_MODEL), lambda i: (i, 0)),
            pl.BlockSpec((TB, D1), lambda i: (i, 0)),
            pl.BlockSpec((D_MODEL, D_FF), lambda i: (0, 0)),
            pl.BlockSpec((D_FF, D_MODEL), lambda i: (0, 0)),
            pl.BlockSpec((1, D_MODEL), lambda i: (0, 0)),
        ],
        out_specs=pl.BlockSpec((TB, D_MODEL), lambda i: (i, 0)),
        out_shape=jax.ShapeDtypeStruct((T, D_MODEL), jnp.float32),
    )(xb, am, w1f, w2f, b2f)
    return out.reshape(B, S, D_MODEL)


# speedup vs baseline: 1.1052x; 1.1052x over previous
import jax
import jax.numpy as jnp
from jax.experimental import pallas as pl

D_MODEL = 1024
D_FF = 4096
N_BLOCK = 32
D1 = 128
D_LOWRANK = 64
TB = 512
TBC = 1024


def _ctrl_body(x_ref, m1_ref, m2_ref, mb_ref, am_ref):
    xt = x_ref[...]
    t1 = jnp.dot(xt, m1_ref[...], preferred_element_type=jnp.float32)
    lg = jnp.dot(t1, m2_ref[...], preferred_element_type=jnp.float32)
    lg = lg + mb_ref[...]
    m = lg[:, 0:D1]
    am = jnp.zeros((TBC, D1), dtype=jnp.int32)
    for y in range(1, N_BLOCK):
        ly = lg[:, y * D1:(y + 1) * D1]
        gt = ly > m
        am = jnp.where(gt, y, am)
        m = jnp.where(gt, ly, m)
    am_ref[...] = am


def _main_body(x_ref, am_ref, w1_ref, w2_ref, b2_ref, out_ref):
    xt = x_ref[...]
    am = am_ref[...]
    midf = jnp.dot(xt, w1_ref[...], preferred_element_type=jnp.float32)
    zeros = jnp.zeros((TB, D1), dtype=jnp.float32)
    pieces = []
    for y in range(N_BLOCK):
        my = midf[:, y * D1:(y + 1) * D1]
        pieces.append(jnp.where(am == y, jnp.maximum(my, 0.0), zeros))
    hsel = jnp.concatenate(pieces, axis=1)
    out = jnp.dot(hsel, w2_ref[...], preferred_element_type=jnp.float32)
    out_ref[...] = out + b2_ref[...]


@jax.jit
def kernel(x, m1, m2, mb, w1, w2, b2):
    B, S, _ = x.shape
    T = B * S
    xf = x.reshape(T, D_MODEL)
    m2f = m2.transpose(0, 2, 1).reshape(D_LOWRANK, D_FF)
    mbf = mb.transpose(1, 0).reshape(1, D_FF)
    w1f = w1.transpose(0, 2, 1).reshape(D_MODEL, D_FF)
    w2f = w2.reshape(D_FF, D_MODEL)
    b2f = b2.reshape(1, D_MODEL)

    grid = (T // TB,)
    am = pl.pallas_call(
        _ctrl_body,
        grid=(T // TBC,),
        in_specs=[
            pl.BlockSpec((TBC, D_MODEL), lambda i: (i, 0)),
            pl.BlockSpec((D_MODEL, D_LOWRANK), lambda i: (0, 0)),
            pl.BlockSpec((D_LOWRANK, D_FF), lambda i: (0, 0)),
            pl.BlockSpec((1, D_FF), lambda i: (0, 0)),
        ],
        out_specs=pl.BlockSpec((TBC, D1), lambda i: (i, 0)),
        out_shape=jax.ShapeDtypeStruct((T, D1), jnp.int32),
    )(xf, m1, m2f, mbf)
    out = pl.pallas_call(
        _main_body,
        grid=grid,
        in_specs=[
            pl.BlockSpec((TB, D_MODEL), lambda i: (i, 0)),
            pl.BlockSpec((TB, D1), lambda i: (i, 0)),
            pl.BlockSpec((D_MODEL, D_FF), lambda i: (0, 0)),
            pl.BlockSpec((D_FF, D_MODEL), lambda i: (0, 0)),
            pl.BlockSpec((1, D_MODEL), lambda i: (0, 0)),
        ],
        out_specs=pl.BlockSpec((TB, D_MODEL), lambda i: (i, 0)),
        out_shape=jax.ShapeDtypeStruct((T, D_MODEL), jnp.float32),
    )(xf, am, w1f, w2f, b2f)
    return out.reshape(B, S, D_MODEL)
